# baseline (speedup 1.0000x reference)
import functools

import jax
import jax.numpy as jnp
from jax import lax
from jax.experimental import pallas as pl
from jax.experimental.pallas import tpu as pltpu
from jax.experimental.pallas import tpu_sc as plsc

BATCH = 16384
HD = 32
NLEV = 4
EMBED = NLEV * HD
NC = 2
NS = 16
NW = NC * NS
BPW = BATCH // NW
CHUNK = 128
NCH = BPW // CHUNK


def _mesh():
    return plsc.VectorSubcoreMesh(core_axis_name="c", subcore_axis_name="s")


@functools.partial(
    pl.kernel,
    out_type=jax.ShapeDtypeStruct((BATCH, EMBED), jnp.float32),
    mesh=_mesh(),
    scratch_types=[
        pltpu.VMEM((NLEV * NCH, CHUNK), jnp.int32),
        pltpu.VMEM((BPW, EMBED), jnp.float32),
        pltpu.SemaphoreType.DMA,
    ],
)
def _sc_embed(i0, i1, i2, i3, w0, w1, w2, w3, out_hbm, idx_v, out_v, sem):
    wid = lax.axis_index("s") * NC + lax.axis_index("c")
    base = wid * BPW
    idx_hbm = (i0, i1, i2, i3)
    tables = (w0, w1, w2, w3)
    row0 = wid * NCH
    for l in range(NLEV):
        pltpu.sync_copy(idx_hbm[l].at[pl.ds(row0, NCH)],
                        idx_v.at[pl.ds(l * NCH, NCH)])
    phase1 = [
        pltpu.async_copy(tables[0].at[idx_v.at[j]],
                         out_v.at[pl.ds(j * CHUNK, CHUNK)], sem)
        for j in range(NCH)
    ]
    for cp in phase1:
        cp.wait()
    phase2 = [
        pltpu.async_copy(tables[l].at[idx_v.at[l * NCH + j]],
                         out_v.at[pl.ds(j * CHUNK, CHUNK)], sem, add=True)
        for l in range(1, NLEV)
        for j in range(NCH)
    ]
    for cp in phase2:
        cp.wait()
    pltpu.sync_copy(out_v, out_hbm.at[pl.ds(base, BPW)])


def kernel(idx0, idx1, idx2, idx3, W0, W1, W2, W3):
    shaped = [
        jnp.reshape(i, (BATCH // CHUNK, CHUNK)).astype(jnp.int32)
        for i in (idx0, idx1, idx2, idx3)
    ]
    padded = [
        jnp.pad(w, ((0, 0), (l * HD, EMBED - (l + 1) * HD)))
        for l, w in enumerate((W0, W1, W2, W3))
    ]
    out = _sc_embed(*shaped, *padded)
    return jnp.reshape(out, (BATCH, 1, EMBED))


# speedup vs baseline: 3.6924x; 3.6924x over previous
import functools

import jax
import jax.numpy as jnp
from jax import lax
from jax.experimental import pallas as pl
from jax.experimental.pallas import tpu as pltpu
from jax.experimental.pallas import tpu_sc as plsc

BATCH = 16384
HD = 32
NLEV = 4
EMBED = NLEV * HD
NCLS = (3, 6, 9, 38)
NC = 2
NS = 16
NW = NC * NS
BPW = BATCH // NW
CHUNK = 128
NCH = BPW // CHUNK
L = 16


def _mesh():
    return plsc.VectorSubcoreMesh(core_axis_name="c", subcore_axis_name="s")


@functools.partial(
    pl.kernel,
    out_type=jax.ShapeDtypeStruct((BATCH * EMBED,), jnp.float32),
    mesh=_mesh(),
    compiler_params=pltpu.CompilerParams(needs_layout_passes=False),
    scratch_types=[
        pltpu.VMEM((NLEV * NCH, CHUNK), jnp.int32),
        [pltpu.VMEM((n * HD,), jnp.float32) for n in NCLS],
        pltpu.VMEM((BPW * EMBED,), jnp.float32),
    ],
)
def _sc_embed(i0, i1, i2, i3, w0, w1, w2, w3, out_hbm, idx_v, tabs_v, out_v):
    wid = lax.axis_index("s") * NC + lax.axis_index("c")
    base = wid * BPW
    idx_hbm = (i0, i1, i2, i3)
    tabs_hbm = (w0, w1, w2, w3)
    for l in range(NLEV):
        pltpu.sync_copy(tabs_hbm[l], tabs_v[l])
        pltpu.sync_copy(idx_hbm[l].at[pl.ds(wid * NCH, NCH)],
                        idx_v.at[pl.ds(l * NCH, NCH)])
    lanes = lax.iota(jnp.int32, L)
    for l in range(NLEV):
        for j in range(NCH):
            def body(g, _, l=l, j=j):
                iv = idx_v[l * NCH + j, pl.ds(g * L, L)]
                rows = iv * HD
                pos0 = (j * CHUNK + g * L + lanes) * EMBED + l * HD
                for d in range(HD):
                    val = plsc.load_gather(tabs_v[l], [rows + d])
                    plsc.store_scatter(out_v, [pos0 + d], val)
                return 0
            lax.fori_loop(0, CHUNK // L, body, 0)
    pltpu.sync_copy(out_v, out_hbm.at[pl.ds(base * EMBED, BPW * EMBED)])


def kernel(idx0, idx1, idx2, idx3, W0, W1, W2, W3):
    shaped = [
        jnp.reshape(i, (BATCH // CHUNK, CHUNK)).astype(jnp.int32)
        for i in (idx0, idx1, idx2, idx3)
    ]
    flat_tabs = [jnp.reshape(w, (-1,)) for w in (W0, W1, W2, W3)]
    out = _sc_embed(*shaped, *flat_tabs)
    return jnp.reshape(out, (BATCH, 1, EMBED))


# speedup vs baseline: 4.9489x; 1.3403x over previous
import functools

import jax
import jax.numpy as jnp
from jax import lax
from jax.experimental import pallas as pl
from jax.experimental.pallas import tpu as pltpu
from jax.experimental.pallas import tpu_sc as plsc

BATCH = 16384
HD = 32
NLEV = 4
EMBED = NLEV * HD
NCLS = (3, 6, 9, 38)
NC = 2
NS = 16
NW = NC * NS
BPW = BATCH // NW
CHUNK = 128
NCH = BPW // CHUNK
L = 16


def _mesh():
    return plsc.VectorSubcoreMesh(core_axis_name="c", subcore_axis_name="s")


@functools.partial(
    pl.kernel,
    out_type=jax.ShapeDtypeStruct((BATCH * EMBED,), jnp.float32),
    mesh=_mesh(),
    compiler_params=pltpu.CompilerParams(needs_layout_passes=False),
    scratch_types=[
        pltpu.VMEM((NLEV * NCH, CHUNK), jnp.int32),
        [pltpu.VMEM((n * HD,), jnp.float32) for n in NCLS],
        pltpu.VMEM((BPW * EMBED,), jnp.float32),
    ],
)
def _sc_embed(i0, i1, i2, i3, w0, w1, w2, w3, out_hbm, idx_v, tabs_v, out_v):
    wid = lax.axis_index("s") * NC + lax.axis_index("c")
    base = wid * BPW
    idx_hbm = (i0, i1, i2, i3)
    tabs_hbm = (w0, w1, w2, w3)
    for l in range(NLEV):
        pltpu.sync_copy(tabs_hbm[l], tabs_v[l])
        pltpu.sync_copy(idx_hbm[l].at[pl.ds(wid * NCH, NCH)],
                        idx_v.at[pl.ds(l * NCH, NCH)])
    lanes = lax.iota(jnp.int32, L)
    for l in range(NLEV):
        for j in range(NCH):
            @plsc.parallel_loop(0, CHUNK // L)
            def body(g, l=l, j=j):
                iv = idx_v[l * NCH + j, pl.ds(g * L, L)]
                rows = iv * HD
                pos0 = (j * CHUNK + g * L + lanes) * EMBED + l * HD
                for d in range(HD):
                    val = plsc.load_gather(tabs_v[l], [rows + d])
                    plsc.store_scatter(out_v, [pos0 + d], val)
    pltpu.sync_copy(out_v, out_hbm.at[pl.ds(base * EMBED, BPW * EMBED)])


def kernel(idx0, idx1, idx2, idx3, W0, W1, W2, W3):
    shaped = [
        jnp.reshape(i, (BATCH // CHUNK, CHUNK)).astype(jnp.int32)
        for i in (idx0, idx1, idx2, idx3)
    ]
    flat_tabs = [jnp.reshape(w, (-1,)) for w in (W0, W1, W2, W3)]
    out = _sc_embed(*shaped, *flat_tabs)
    return jnp.reshape(out, (BATCH, 1, EMBED))


# speedup vs baseline: 10.2211x; 2.0653x over previous
import functools

import jax
import jax.numpy as jnp
from jax import lax
from jax.experimental import pallas as pl
from jax.experimental.pallas import tpu as pltpu
from jax.experimental.pallas import tpu_sc as plsc

BATCH = 16384
HD = 32
NLEV = 4
EMBED = NLEV * HD
NCLS = (3, 6, 9, 38)
NC = 2
NS = 16
NW = NC * NS
BPW = BATCH // NW
CHUNK = 128
NCH = BPW // CHUNK
L = 16


def _mesh():
    return plsc.VectorSubcoreMesh(core_axis_name="c", subcore_axis_name="s")


@functools.partial(
    pl.kernel,
    out_type=jax.ShapeDtypeStruct((BATCH * EMBED,), jnp.float32),
    mesh=_mesh(),
    compiler_params=pltpu.CompilerParams(needs_layout_passes=False),
    scratch_types=[
        pltpu.VMEM((NLEV * NCH, CHUNK), jnp.int32),
        [pltpu.VMEM((n * HD,), jnp.float32) for n in NCLS],
        pltpu.VMEM((BPW * EMBED,), jnp.float32),
    ],
)
def _sc_embed(i0, i1, i2, i3, w0, w1, w2, w3, out_hbm, idx_v, tabs_v, out_v):
    wid = lax.axis_index("s") * NC + lax.axis_index("c")
    base = wid * BPW
    idx_hbm = (i0, i1, i2, i3)
    tabs_hbm = (w0, w1, w2, w3)
    for l in range(NLEV):
        pltpu.sync_copy(tabs_hbm[l], tabs_v[l])
        pltpu.sync_copy(idx_hbm[l].at[pl.ds(wid * NCH, NCH)],
                        idx_v.at[pl.ds(l * NCH, NCH)])
    @plsc.parallel_loop(0, BPW // L, unroll=2)
    def body(g):
        row = g >> 3
        col0 = (g & 7) * L
        obase = pl.multiple_of(g * L * EMBED, L * EMBED)
        for l in range(NLEV):
            iv = idx_v[l * NCH + row, pl.ds(col0, L)]
            for i in range(L):
                roff = iv[i] * HD
                for k in range(HD // L):
                    src = pl.multiple_of(roff + k * L, L)
                    out_v[pl.ds(obase + i * EMBED + l * HD + k * L, L)] = (
                        tabs_v[l][pl.ds(src, L)])
    pltpu.sync_copy(out_v, out_hbm.at[pl.ds(base * EMBED, BPW * EMBED)])


def kernel(idx0, idx1, idx2, idx3, W0, W1, W2, W3):
    shaped = [
        jnp.reshape(i, (BATCH // CHUNK, CHUNK)).astype(jnp.int32)
        for i in (idx0, idx1, idx2, idx3)
    ]
    flat_tabs = [jnp.reshape(w, (-1,)) for w in (W0, W1, W2, W3)]
    out = _sc_embed(*shaped, *flat_tabs)
    return jnp.reshape(out, (BATCH, 1, EMBED))
